# baseline (speedup 1.0000x reference)
import functools

import jax
import jax.numpy as jnp
from jax import lax
from jax.experimental import pallas as pl
from jax.experimental.pallas import tpu as pltpu
from jax.experimental.pallas import tpu_sc as plsc

_VOCAB = 1000000
_D = 64
_BATCH = 16384
_HIST = 50
_TOTAL = _BATCH * _HIST
_NC = 2
_NS = 16
_NW = _NC * _NS
_PER_W = _TOTAL // _NW
_CH = 800
_NCHUNK = _PER_W // _CH

_mesh = plsc.VectorSubcoreMesh(core_axis_name="c", subcore_axis_name="s")


@functools.partial(
    pl.kernel,
    mesh=_mesh,
    out_type=jax.ShapeDtypeStruct((_TOTAL, _D), jnp.float32),
    scratch_types=[
        pltpu.VMEM((_PER_W,), jnp.int32),
        pltpu.VMEM((_CH, _D), jnp.float32),
        pltpu.VMEM((_CH, _D), jnp.float32),
        pltpu.SemaphoreType.DMA,
        pltpu.SemaphoreType.DMA,
    ],
    compiler_params=pltpu.CompilerParams(use_tc_tiling_on_sc=False),
)
def _gather_kernel(idx2_hbm, table2m_hbm, out_hbm, idx_v, buf0, buf1, g0, g1):
    wid = lax.axis_index("s") * _NC + lax.axis_index("c")
    base = wid * _PER_W
    pltpu.sync_copy(idx2_hbm.at[pl.ds(base, _PER_W)], idx_v)

    def gather_start(c, buf, sem):
        pltpu.async_copy(table2m_hbm.at[idx_v.at[pl.ds(c * _CH, _CH)]], buf, sem)

    def gather_wait(c, buf, sem):
        pltpu.make_async_copy(
            table2m_hbm.at[idx_v.at[pl.ds(c * _CH, _CH)]], buf, sem
        ).wait()

    gather_start(0, buf0, g0)
    gather_start(1, buf1, g1)

    def handle(c, buf, sem):
        gather_wait(c, buf, sem)
        pltpu.sync_copy(buf, out_hbm.at[pl.ds(base + c * _CH, _CH)])

        @pl.when(c + 2 < _NCHUNK)
        def _():
            gather_start(c + 2, buf, sem)

    def body(p, carry):
        handle(2 * p, buf0, g0)
        handle(2 * p + 1, buf1, g1)
        return carry

    lax.fori_loop(0, _NCHUNK // 2, body, 0)


def kernel(input, table):
    idx2 = input.reshape(_TOTAL) * 2
    table2m = jnp.pad(table, ((0, 0), (0, 128 - _D))).reshape(2 * _VOCAB, _D)
    out = _gather_kernel(idx2, table2m)
    return out.reshape(_TOTAL // 2, 2 * _D).reshape(_BATCH, _HIST, _D)


# speedup vs baseline: 1.3905x; 1.3905x over previous
import functools

import jax
import jax.numpy as jnp
from jax import lax
from jax.experimental import pallas as pl
from jax.experimental.pallas import tpu as pltpu
from jax.experimental.pallas import tpu_sc as plsc

_VOCAB = 1000000
_D = 64
_BATCH = 16384
_HIST = 50
_TOTAL = _BATCH * _HIST
_NC = 2
_NS = 16
_NW = _NC * _NS
_PER_W = _TOTAL // _NW
_CH = 512
_NCHUNK = _PER_W // _CH

_mesh = plsc.VectorSubcoreMesh(core_axis_name="c", subcore_axis_name="s")


@functools.partial(
    pl.kernel,
    mesh=_mesh,
    out_type=jax.ShapeDtypeStruct((_TOTAL, _D), jnp.float32),
    scratch_types=[
        pltpu.VMEM((_PER_W,), jnp.int32),
        pltpu.VMEM((_CH, _D), jnp.float32),
        pltpu.VMEM((_CH, _D), jnp.float32),
        pltpu.SemaphoreType.DMA,
        pltpu.SemaphoreType.DMA,
    ],
    compiler_params=pltpu.CompilerParams(use_tc_tiling_on_sc=False),
)
def _gather_kernel(idx2_hbm, table2m_hbm, out_hbm, idx_v, buf0, buf1, g0, g1):
    wid = lax.axis_index("s") * _NC + lax.axis_index("c")
    base = wid * _PER_W
    pltpu.sync_copy(idx2_hbm.at[pl.ds(base, _PER_W)], idx_v)

    def gather_start(c, buf, sem):
        pltpu.async_copy(table2m_hbm.at[idx_v.at[pl.ds(c * _CH, _CH)]], buf, sem)

    def gather_wait(c, buf, sem):
        pltpu.make_async_copy(
            table2m_hbm.at[idx_v.at[pl.ds(c * _CH, _CH)]], buf, sem
        ).wait()

    gather_start(0, buf0, g0)
    gather_start(1, buf1, g1)

    def handle(c, buf, sem):
        gather_wait(c, buf, sem)
        pltpu.sync_copy(buf, out_hbm.at[pl.ds(base + c * _CH, _CH)])

        @pl.when(c + 2 < _NCHUNK)
        def _():
            gather_start(c + 2, buf, sem)

    def body(p, carry):
        handle(2 * p, buf0, g0)
        handle(2 * p + 1, buf1, g1)
        return carry

    lax.fori_loop(0, _NCHUNK // 2, body, 0)


_HH = _HIST // 2
_TD = _HIST * _D
_BK = 2048


def _pad_transpose_kernel(in_ref, out_ref):
    x = in_ref[...]
    out_ref[...] = jnp.concatenate([x, x], axis=0).T


def _to_batch_minor_kernel(in_ref, out_ref):
    x = in_ref[...].reshape(128, _HH, 128)
    for s in range(_HH):
        out_ref[pl.ds(s * 128, 128), :] = x[:, s, :].T


def kernel(input, table):
    idx2 = input.reshape(_TOTAL) * 2
    t128 = pl.pallas_call(
        _pad_transpose_kernel,
        grid=(_VOCAB // _BK,),
        in_specs=[pl.BlockSpec((_D, _BK), lambda i: (0, i))],
        out_specs=pl.BlockSpec((_BK, 2 * _D), lambda i: (i, 0)),
        out_shape=jax.ShapeDtypeStruct((_VOCAB, 2 * _D), jnp.float32),
    )(table.T)
    table2m = t128.reshape(2 * _VOCAB, _D)
    out = _gather_kernel(idx2, table2m)
    g128 = out.reshape(_TOTAL // 2, 2 * _D)
    o2 = pl.pallas_call(
        _to_batch_minor_kernel,
        grid=(_BATCH // 128,),
        in_specs=[pl.BlockSpec((_HH * 128, 128), lambda i: (i, 0))],
        out_specs=pl.BlockSpec((_TD, 128), lambda i: (0, i)),
        out_shape=jax.ShapeDtypeStruct((_TD, _BATCH), jnp.float32),
    )(g128)
    return o2.reshape(_HIST, _D, _BATCH).transpose(2, 0, 1)


# speedup vs baseline: 1.5005x; 1.0791x over previous
import functools

import jax
import jax.numpy as jnp
from jax import lax
from jax.experimental import pallas as pl
from jax.experimental.pallas import tpu as pltpu
from jax.experimental.pallas import tpu_sc as plsc

_VOCAB = 1000000
_D = 64
_BATCH = 16384
_HIST = 50
_TOTAL = _BATCH * _HIST
_NC = 2
_NS = 16
_NW = _NC * _NS
_PER_W = _TOTAL // _NW
_CH = 512
_NCHUNK = _PER_W // _CH

_mesh = plsc.VectorSubcoreMesh(core_axis_name="c", subcore_axis_name="s")


@functools.partial(
    pl.kernel,
    mesh=_mesh,
    out_type=jax.ShapeDtypeStruct((_TOTAL, _D), jnp.float32),
    scratch_types=[
        pltpu.VMEM((_PER_W,), jnp.int32),
        pltpu.VMEM((_CH, _D), jnp.float32),
        pltpu.VMEM((_CH, _D), jnp.float32),
        pltpu.SemaphoreType.DMA,
        pltpu.SemaphoreType.DMA,
    ],
    compiler_params=pltpu.CompilerParams(use_tc_tiling_on_sc=False),
)
def _gather_kernel(idx2_hbm, table2m_hbm, out_hbm, idx_v, buf0, buf1, g0, g1):
    wid = lax.axis_index("s") * _NC + lax.axis_index("c")
    base = wid * _PER_W
    pltpu.sync_copy(idx2_hbm.at[pl.ds(base, _PER_W)], idx_v)

    def gather_start(c, buf, sem):
        pltpu.async_copy(table2m_hbm.at[idx_v.at[pl.ds(c * _CH, _CH)]], buf, sem)

    def gather_wait(c, buf, sem):
        pltpu.make_async_copy(
            table2m_hbm.at[idx_v.at[pl.ds(c * _CH, _CH)]], buf, sem
        ).wait()

    gather_start(0, buf0, g0)
    gather_start(1, buf1, g1)

    def handle(c, buf, sem):
        gather_wait(c, buf, sem)
        pltpu.sync_copy(buf, out_hbm.at[pl.ds(base + c * _CH, _CH)])

        @pl.when(c + 2 < _NCHUNK)
        def _():
            gather_start(c + 2, buf, sem)

    def body(p, carry):
        handle(2 * p, buf0, g0)
        handle(2 * p + 1, buf1, g1)
        return carry

    lax.fori_loop(0, _NCHUNK // 2, body, 0)


_HH = _HIST // 2
_TD = _HIST * _D
_BK = 2048


def _pad_transpose_kernel(in_ref, out_ref):
    x = in_ref[...]
    out_ref[...] = jnp.concatenate([x, x], axis=0).T


def _to_batch_minor_kernel(in_ref, out_ref):
    x = in_ref[...].reshape(128, _HH, 128)
    for s in range(_HH):
        out_ref[pl.ds(s * 128, 128), :] = x[:, s, :].T


def kernel(input, table):
    idx2 = input.reshape(_TOTAL) * 2
    t128 = pl.pallas_call(
        _pad_transpose_kernel,
        grid=(pl.cdiv(_VOCAB, _BK),),
        in_specs=[pl.BlockSpec((_D, _BK), lambda i: (0, i))],
        out_specs=pl.BlockSpec((_BK, 2 * _D), lambda i: (i, 0)),
        out_shape=jax.ShapeDtypeStruct((_VOCAB, 2 * _D), jnp.float32),
    )(table.T)
    table2m = t128.reshape(2 * _VOCAB, _D)
    out = _gather_kernel(idx2, table2m)
    g128 = out.reshape(_TOTAL // 2, 2 * _D)
    o2 = pl.pallas_call(
        _to_batch_minor_kernel,
        grid=(_BATCH // 128,),
        in_specs=[pl.BlockSpec((_HH * 128, 128), lambda i: (i, 0))],
        out_specs=pl.BlockSpec((_TD, 128), lambda i: (0, i)),
        out_shape=jax.ShapeDtypeStruct((_TD, _BATCH), jnp.float32),
    )(g128)
    return o2.reshape(_HIST, _D, _BATCH).transpose(2, 0, 1)
